# baseline (speedup 1.0000x reference)
import jax
import jax.numpy as jnp
from jax import lax
from jax.experimental import pallas as pl
from jax.experimental.pallas import tpu as pltpu
from jax.experimental.pallas import tpu_sc as plsc

NUM_USER = 1000000
EMBED_DIM = 32
BATCH = 16384


def kernel(user_fea, table):
    idx = user_fea[:, 0].astype(jnp.int32)
    flat_t = table.T.reshape(-1)

    info = plsc.get_sparse_core_info()
    nc, ns, nl = info.num_cores, info.num_subcores, info.num_lanes
    nw = nc * ns
    b_per_w = BATCH // nw
    slab = EMBED_DIM * b_per_w

    def body(flat_hbm, idx_hbm, out_hbm, idx_v, widx, gath, sem):
        wid = lax.axis_index("s") * nc + lax.axis_index("c")
        base = wid * b_per_w
        pltpu.sync_copy(idx_hbm.at[pl.ds(base, b_per_w)], idx_v)
        for g in range(b_per_w // nl):
            vec = idx_v[pl.ds(g * nl, nl)]
            for c in range(EMBED_DIM):
                widx[pl.ds(c * b_per_w + g * nl, nl)] = vec + c * NUM_USER
        pltpu.async_copy(flat_hbm.at[widx], gath, sem).wait()
        pltpu.sync_copy(gath, out_hbm.at[pl.ds(wid * slab, slab)])

    f = pl.kernel(
        body,
        out_type=jax.ShapeDtypeStruct((BATCH * EMBED_DIM,), jnp.float32),
        mesh=plsc.VectorSubcoreMesh(core_axis_name="c", subcore_axis_name="s"),
        scratch_types=[
            pltpu.VMEM((b_per_w,), jnp.int32),
            pltpu.VMEM((slab,), jnp.int32),
            pltpu.VMEM((slab,), jnp.float32),
            pltpu.SemaphoreType.DMA,
        ],
    )
    out_flat = f(flat_t, idx)
    out_t = (
        out_flat.reshape(nw, EMBED_DIM, b_per_w)
        .transpose(1, 0, 2)
        .reshape(EMBED_DIM, BATCH)
    )
    return out_t.T


# speedup vs baseline: 4.9316x; 4.9316x over previous
import jax
import jax.numpy as jnp
from jax import lax
from jax.experimental import pallas as pl
from jax.experimental.pallas import tpu as pltpu
from jax.experimental.pallas import tpu_sc as plsc

NUM_USER = 1000000
EMBED_DIM = 32
BATCH = 16384


def kernel(user_fea, table):
    idx = user_fea[:, 0].astype(jnp.int32)

    info = plsc.get_sparse_core_info()
    nc, ns = info.num_cores, info.num_subcores
    nw = nc * ns
    b_per_w = BATCH // nw

    def body(table_hbm, idx_hbm, out_hbm, idx_v, rows_v, sem):
        wid = lax.axis_index("s") * nc + lax.axis_index("c")
        base = wid * b_per_w
        pltpu.sync_copy(idx_hbm.at[pl.ds(base, b_per_w)], idx_v)
        pltpu.async_copy(table_hbm.at[idx_v], rows_v, sem).wait()
        pltpu.sync_copy(rows_v, out_hbm.at[pl.ds(base, b_per_w)])

    f = pl.kernel(
        body,
        out_type=jax.ShapeDtypeStruct((BATCH, EMBED_DIM), jnp.float32),
        mesh=plsc.VectorSubcoreMesh(core_axis_name="c", subcore_axis_name="s"),
        scratch_types=[
            pltpu.VMEM((b_per_w,), jnp.int32),
            pltpu.VMEM((b_per_w, EMBED_DIM), jnp.float32),
            pltpu.SemaphoreType.DMA,
        ],
        compiler_params=pltpu.CompilerParams(use_tc_tiling_on_sc=False),
    )
    return f(table, idx)


# speedup vs baseline: 17.8087x; 3.6112x over previous
import jax
import jax.numpy as jnp
from jax import lax
from jax.experimental import pallas as pl
from jax.experimental.pallas import tpu as pltpu
from jax.experimental.pallas import tpu_sc as plsc

NUM_USER = 1000000
EMBED_DIM = 32
BATCH = 16384
LANES = 16


def kernel(user_fea, table):
    idx = user_fea[:, 0].astype(jnp.int32)
    table_t = table.T

    info = plsc.get_sparse_core_info()
    nc, ns = info.num_cores, info.num_subcores
    nw = nc * ns
    b_per_w = BATCH // nw
    n_batches = b_per_w // LANES

    def body(tt_hbm, idx_hbm, out_hbm, idx_v, ring, slab, sem):
        wid = lax.axis_index("s") * nc + lax.axis_index("c")
        base = pl.multiple_of(wid * b_per_w, b_per_w)
        pltpu.sync_copy(idx_hbm.at[pl.ds(base, b_per_w)], idx_v)
        lanes = lax.iota(jnp.int32, LANES)

        def batch(g, carry):
            vec = idx_v[pl.ds(g * LANES, LANES)]
            ms = []
            for k in range(LANES):
                r = jnp.sum(jnp.where(lanes == k, vec, 0))
                q = pl.multiple_of((r >> 7) * 128, 128)
                ms.append(r & 127)
                pltpu.async_copy(tt_hbm.at[:, pl.ds(q, 128)], ring.at[k], sem)
            for k in range(LANES):
                pltpu.make_async_copy(
                    tt_hbm.at[:, pl.ds(0, 128)], ring.at[k], sem
                ).wait()
            jloc = g * LANES
            for k in range(LANES):
                mvec = lanes * 0 + ms[k]
                jvec = lanes * 0 + (jloc + k)
                lo = plsc.load_gather(ring.at[k], [lanes, mvec])
                hi = plsc.load_gather(ring.at[k], [lanes + LANES, mvec])
                plsc.store_scatter(slab, [lanes, jvec], lo)
                plsc.store_scatter(slab, [lanes + LANES, jvec], hi)
            return carry

        lax.fori_loop(0, n_batches, batch, 0)
        pltpu.sync_copy(slab, out_hbm.at[:, pl.ds(base, b_per_w)])

    f = pl.kernel(
        body,
        out_type=jax.ShapeDtypeStruct((EMBED_DIM, BATCH), jnp.float32),
        mesh=plsc.VectorSubcoreMesh(core_axis_name="c", subcore_axis_name="s"),
        scratch_types=[
            pltpu.VMEM((b_per_w,), jnp.int32),
            pltpu.VMEM((LANES, EMBED_DIM, 128), jnp.float32),
            pltpu.VMEM((EMBED_DIM, b_per_w), jnp.float32),
            pltpu.SemaphoreType.DMA,
        ],
        compiler_params=pltpu.CompilerParams(
            use_tc_tiling_on_sc=True, needs_layout_passes=False
        ),
    )
    out_t = f(table_t, idx)
    return out_t.T


# speedup vs baseline: 18.9325x; 1.0631x over previous
import jax
import jax.numpy as jnp
from jax import lax
from jax.experimental import pallas as pl
from jax.experimental.pallas import tpu as pltpu
from jax.experimental.pallas import tpu_sc as plsc

NUM_USER = 1000000
EMBED_DIM = 32
BATCH = 16384
LANES = 16
BSZ = 8


def kernel(user_fea, table):
    idx = user_fea[:, 0].astype(jnp.int32)
    table_t = table.T

    info = plsc.get_sparse_core_info()
    nc, ns = info.num_cores, info.num_subcores
    nw = nc * ns
    b_per_w = BATCH // nw
    n_batches = b_per_w // BSZ

    def body(tt_hbm, idx_hbm, out_hbm, idx_v, ring, slab, sem_a, sem_b):
        wid = lax.axis_index("s") * nc + lax.axis_index("c")
        base = pl.multiple_of(wid * b_per_w, b_per_w)
        pltpu.sync_copy(idx_hbm.at[pl.ds(base, b_per_w)], idx_v.at[pl.ds(0, b_per_w)])
        lanes = lax.iota(jnp.int32, LANES)

        def fire(n, slot0, sem):
            vec = idx_v[pl.ds(n * BSZ, LANES)]
            for k in range(BSZ):
                r = jnp.sum(jnp.where(lanes == k, vec, 0))
                q = pl.multiple_of((r >> 7) * 128, 128)
                pltpu.async_copy(
                    tt_hbm.at[:, pl.ds(q, 128)], ring.at[slot0 + k], sem
                )

        def drain(slot0, sem):
            for k in range(BSZ):
                pltpu.make_async_copy(
                    tt_hbm.at[:, pl.ds(0, 128)], ring.at[slot0 + k], sem
                ).wait()

        def extract(n, slot0):
            vec = idx_v[pl.ds(n * BSZ, LANES)]
            for k in range(BSZ):
                m = jnp.sum(jnp.where(lanes == k, vec, 0)) & 127
                mvec = lanes * 0 + m
                jvec = lanes * 0 + (n * BSZ + k)
                lo = plsc.load_gather(ring.at[slot0 + k], [lanes, mvec])
                hi = plsc.load_gather(ring.at[slot0 + k], [lanes + LANES, mvec])
                plsc.store_scatter(slab, [lanes, jvec], lo)
                plsc.store_scatter(slab, [lanes + LANES, jvec], hi)

        fire(0, 0, sem_a)

        def step(t, carry):
            pl.when(2 * t + 1 < n_batches)(lambda: fire(2 * t + 1, BSZ, sem_b))
            drain(0, sem_a)
            extract(2 * t, 0)
            pl.when(2 * t + 2 < n_batches)(lambda: fire(2 * t + 2, 0, sem_a))
            pl.when(2 * t + 1 < n_batches)(
                lambda: (drain(BSZ, sem_b), extract(2 * t + 1, BSZ))[1]
            )
            return carry

        lax.fori_loop(0, n_batches // 2, step, 0)
        pltpu.sync_copy(slab, out_hbm.at[:, pl.ds(base, b_per_w)])

    f = pl.kernel(
        body,
        out_type=jax.ShapeDtypeStruct((EMBED_DIM, BATCH), jnp.float32),
        mesh=plsc.VectorSubcoreMesh(core_axis_name="c", subcore_axis_name="s"),
        scratch_types=[
            pltpu.VMEM((b_per_w + LANES - BSZ,), jnp.int32),
            pltpu.VMEM((2 * BSZ, EMBED_DIM, 128), jnp.float32),
            pltpu.VMEM((EMBED_DIM, b_per_w), jnp.float32),
            pltpu.SemaphoreType.DMA,
            pltpu.SemaphoreType.DMA,
        ],
        compiler_params=pltpu.CompilerParams(
            use_tc_tiling_on_sc=True, needs_layout_passes=False
        ),
    )
    out_t = f(table_t, idx)
    return out_t.T
